# baseline (speedup 1.0000x reference)
import jax
import jax.numpy as jnp
from jax import lax
from jax.experimental import pallas as pl
from jax.experimental.pallas import tpu as pltpu

_BB = 4


def _body(hw_ref, x_ref, yt_ref, xt_ref, o_ref, pos_ref):
    seq = pos_ref.shape[0]
    n_rows = yt_ref.shape[0]

    @pl.when(pl.program_id(0) == 0)
    def _compute_pos():
        h = hw_ref[0]
        w = hw_ref[1]
        p = lax.broadcasted_iota(jnp.int32, (seq, n_rows), 0)
        j = lax.broadcasted_iota(jnp.int32, (seq, n_rows), 1)
        inv_w = 1.0 / w.astype(jnp.float32)
        r = jnp.floor(p.astype(jnp.float32) * inv_w + 0.0009765625)
        r = r.astype(jnp.int32)
        y_idx = jnp.minimum(r, h - 1)
        x_idx = p - w * r
        oh_y = (y_idx == j).astype(jnp.float32)
        oh_x = (x_idx == j).astype(jnp.float32)
        y_emb = jnp.dot(oh_y, yt_ref[...], preferred_element_type=jnp.float32,
                        precision=lax.Precision.HIGHEST)
        x_emb = jnp.dot(oh_x, xt_ref[...], preferred_element_type=jnp.float32,
                        precision=lax.Precision.HIGHEST)
        pos_ref[...] = jnp.concatenate([y_emb, x_emb], axis=-1)

    @pl.when(pl.program_id(0) > 0)
    def _add():
        pos = pos_ref[...]
        for i in range(o_ref.shape[0]):
            o_ref[i] = x_ref[i] + pos


def kernel(x, y_table, x_table, h, w):
    B, seq, D = x.shape
    hw = jnp.array([h, w], dtype=jnp.int32)

    def xo_map(i, hw_ref):
        return (jnp.maximum(i - 1, 0), 0, 0)

    grid_spec = pltpu.PrefetchScalarGridSpec(
        num_scalar_prefetch=1,
        grid=(B // _BB + 1,),
        in_specs=[
            pl.BlockSpec((_BB, seq, D), xo_map),
            pl.BlockSpec(y_table.shape, lambda i, hw_ref: (0, 0)),
            pl.BlockSpec(x_table.shape, lambda i, hw_ref: (0, 0)),
        ],
        out_specs=pl.BlockSpec((_BB, seq, D), xo_map),
        scratch_shapes=[pltpu.VMEM((seq, D), jnp.float32)],
    )
    return pl.pallas_call(
        _body,
        grid_spec=grid_spec,
        out_shape=jax.ShapeDtypeStruct((B, seq, D), x.dtype),
    )(hw, x, y_table, x_table)


# speedup vs baseline: 1.0368x; 1.0368x over previous
import jax
import jax.numpy as jnp
from jax import lax
from jax.experimental import pallas as pl
from jax.experimental.pallas import tpu as pltpu

_BB = 4


def _body(hw_ref, x_ref, yt_ref, xt_ref, o_ref, pos_ref):
    seq = pos_ref.shape[0]
    n_rows = yt_ref.shape[0]
    half = yt_ref.shape[1]

    @pl.when(pl.program_id(0) == 0)
    def _compute_pos():
        h = hw_ref[0]
        w = hw_ref[1]
        p = lax.broadcasted_iota(jnp.int32, (seq, n_rows), 0)
        j = lax.broadcasted_iota(jnp.int32, (seq, n_rows), 1)
        inv_w = 1.0 / w.astype(jnp.float32)
        r = jnp.floor(p.astype(jnp.float32) * inv_w + 0.0009765625)
        r = r.astype(jnp.int32)
        y_idx = jnp.minimum(r, h - 1)
        x_idx = p - w * r
        oh_y = (y_idx == j).astype(jnp.float32)
        oh_x = (x_idx == j).astype(jnp.float32)

        yt = yt_ref[...]
        xt = xt_ref[...]
        yt_hi = yt.astype(jnp.bfloat16).astype(jnp.float32)
        xt_hi = xt.astype(jnp.bfloat16).astype(jnp.float32)
        z = jnp.zeros((n_rows, half), jnp.float32)
        bd = jnp.concatenate(
            [
                jnp.concatenate([yt_hi, z], axis=1),
                jnp.concatenate([yt - yt_hi, z], axis=1),
                jnp.concatenate([z, xt_hi], axis=1),
                jnp.concatenate([z, xt - xt_hi], axis=1),
            ],
            axis=0,
        )
        oh4 = jnp.concatenate([oh_y, oh_y, oh_x, oh_x], axis=1)
        pos_ref[...] = jnp.dot(oh4, bd, preferred_element_type=jnp.float32)

    pos = pos_ref[...]
    for i in range(o_ref.shape[0]):
        o_ref[i] = x_ref[i] + pos


def kernel(x, y_table, x_table, h, w):
    B, seq, D = x.shape
    hw = jnp.array([h, w], dtype=jnp.int32)

    grid_spec = pltpu.PrefetchScalarGridSpec(
        num_scalar_prefetch=1,
        grid=(B // _BB,),
        in_specs=[
            pl.BlockSpec((_BB, seq, D), lambda b, hw_ref: (b, 0, 0)),
            pl.BlockSpec(y_table.shape, lambda b, hw_ref: (0, 0)),
            pl.BlockSpec(x_table.shape, lambda b, hw_ref: (0, 0)),
        ],
        out_specs=pl.BlockSpec((_BB, seq, D), lambda b, hw_ref: (b, 0, 0)),
        scratch_shapes=[pltpu.VMEM((seq, D), jnp.float32)],
    )
    return pl.pallas_call(
        _body,
        grid_spec=grid_spec,
        out_shape=jax.ShapeDtypeStruct((B, seq, D), x.dtype),
    )(hw, x, y_table, x_table)
